# baseline (speedup 1.0000x reference)
import jax
import jax.numpy as jnp
import numpy as np
from jax import lax
from jax.experimental import pallas as pl
from jax.experimental.pallas import tpu as pltpu
from jax.experimental.pallas import tpu_sc as plsc

_R, _V = 128, 100000
_NT = _V // 8
_CTILE = 17
_NCHUNK = 23
_CVOC = _CTILE * 8
_NW = 32
_NINF = np.float32(-np.inf)

_CONSTS = {}


def _noise_t():
    def make():
        e = jax.random.exponential(jax.random.key(42), (_R, _V), dtype=jnp.float32)
        return jnp.minimum(-jnp.log(e), jnp.float32(3e37))

    if "n" not in _CONSTS:
        try:
            with jax.ensure_compile_time_eval():
                n = np.ascontiguousarray(np.asarray(make()).T)
                _CONSTS["n"] = jnp.asarray(n)
        except Exception:
            return make().T
    return _CONSTS["n"]


def _body(x_hbm, temps_hbm, noise_hbm, ov_hbm, oi_hbm,
          tbuf, lbuf0, lbuf1, nbuf0, nbuf1, vstage, istage, pvstage, pistage,
          sh_v, sh_i, lsem0, lsem1, nsem0, nsem1):
    cid = lax.axis_index("c")
    sid = lax.axis_index("s")
    wid = cid * 16 + sid
    pltpu.sync_copy(temps_hbm, tbuf)
    tvec = [tbuf[pl.ds(16 * k, 16)] for k in range(8)]

    start = 390 * wid + jnp.minimum(wid, 20)
    lbufs, nbufs = (lbuf0, lbuf1), (nbuf0, nbuf1)
    lsems, nsems = (lsem0, lsem1), (nsem0, nsem1)

    def fire(i):
        s = i & 1
        t0 = jnp.minimum(start + _CTILE * i, _NT - _CTILE)
        v0 = pl.multiple_of(t0 * 8, 8)
        return (pltpu.async_copy(x_hbm.at[pl.ds(v0, _CVOC)], lbufs[s], lsems[s]),
                pltpu.async_copy(noise_hbm.at[pl.ds(v0, _CVOC)], nbufs[s], nsems[s]),
                t0)

    best = [jnp.full((16,), _NINF, jnp.float32) for _ in range(8)]
    ci = [jnp.zeros((16,), jnp.int32) for _ in range(8)]

    pend = fire(0)
    for i in range(_NCHUNK):
        nxt = fire(i + 1) if i + 1 < _NCHUNK else None
        pend[0].wait()
        pend[1].wait()
        s = i & 1
        lb, nb = lbufs[s], nbufs[s]
        vbase = pend[2] * 8

        @plsc.parallel_loop(0, _CVOC, 1, unroll=1, carry=tuple(best) + tuple(ci))
        def scan(v, carry, lb=lb, nb=nb, vbase=vbase):
            acc = list(carry)
            vg = jnp.full((16,), vbase + v, jnp.int32)
            for k in range(8):
                val = lb[v, pl.ds(16 * k, 16)] + tvec[k] * nb[v, pl.ds(16 * k, 16)]
                m = val > acc[k]
                acc[k] = jnp.where(m, val, acc[k])
                acc[8 + k] = jnp.where(m, vg, acc[8 + k])
            return tuple(acc)

        best, ci = list(scan[:8]), list(scan[8:])
        pend = nxt

    for k in range(8):
        vstage[pl.ds(16 * k, 16)] = best[k]
        istage[pl.ds(16 * k, 16)] = ci[k]
    pltpu.sync_copy(vstage, sh_v.at[pl.ds(sid * 128, 128)])
    pltpu.sync_copy(istage, sh_i.at[pl.ds(sid * 128, 128)])
    plsc.subcore_barrier()

    @pl.when(sid == 0)
    def _():
        mv = [jnp.full((16,), _NINF, jnp.float32) for _ in range(8)]
        mi = [jnp.zeros((16,), jnp.int32) for _ in range(8)]
        for w in range(16):
            pltpu.sync_copy(sh_v.at[pl.ds(w * 128, 128)], pvstage)
            pltpu.sync_copy(sh_i.at[pl.ds(w * 128, 128)], pistage)
            for k in range(8):
                bv = pvstage[pl.ds(16 * k, 16)]
                bi = pistage[pl.ds(16 * k, 16)]
                take = (bv > mv[k]) | ((bv == mv[k]) & (bi < mi[k]))
                mv[k] = jnp.where(take, bv, mv[k])
                mi[k] = jnp.where(take, bi, mi[k])
        for k in range(8):
            vstage[pl.ds(16 * k, 16)] = mv[k]
            istage[pl.ds(16 * k, 16)] = mi[k]
        pltpu.sync_copy(vstage, ov_hbm.at[pl.ds(cid * 128, 128)])
        pltpu.sync_copy(istage, oi_hbm.at[pl.ds(cid * 128, 128)])


@jax.jit
def _sampler(x_t, temps, noise_t):
    mesh = plsc.VectorSubcoreMesh(core_axis_name="c", subcore_axis_name="s")
    k = pl.kernel(
        _body,
        out_type=(jax.ShapeDtypeStruct((2 * _R,), jnp.float32),
                  jax.ShapeDtypeStruct((2 * _R,), jnp.int32)),
        mesh=mesh,
        compiler_params=pltpu.CompilerParams(needs_layout_passes=False),
        scratch_types=[
            pltpu.VMEM((_R,), jnp.float32),
            pltpu.VMEM((_CVOC, _R), jnp.float32),
            pltpu.VMEM((_CVOC, _R), jnp.float32),
            pltpu.VMEM((_CVOC, _R), jnp.float32),
            pltpu.VMEM((_CVOC, _R), jnp.float32),
            pltpu.VMEM((_R,), jnp.float32),
            pltpu.VMEM((_R,), jnp.int32),
            pltpu.VMEM((_R,), jnp.float32),
            pltpu.VMEM((_R,), jnp.int32),
            pltpu.VMEM_SHARED((16 * _R,), jnp.float32),
            pltpu.VMEM_SHARED((16 * _R,), jnp.int32),
            pltpu.SemaphoreType.DMA,
            pltpu.SemaphoreType.DMA,
            pltpu.SemaphoreType.DMA,
            pltpu.SemaphoreType.DMA,
        ],
    )
    vals, idxs = k(x_t, temps, noise_t)
    v0, v1 = vals[:_R], vals[_R:]
    i0, i1 = idxs[:_R], idxs[_R:]
    take1 = (v1 > v0) | ((v1 == v0) & (i1 < i0))
    return jnp.where(take1, i1, i0)


def kernel(logits, temperatures):
    return _sampler(logits.T, temperatures, _noise_t())


# speedup vs baseline: 1.0053x; 1.0053x over previous
import jax
import jax.numpy as jnp
import numpy as np
from jax import lax
from jax.experimental import pallas as pl
from jax.experimental.pallas import tpu as pltpu
from jax.experimental.pallas import tpu_sc as plsc

_R, _V = 128, 100000
_NT = _V // 8
_CTILE = 23
_NCHUNK = 17
_CVOC = _CTILE * 8
_NW = 32
_NINF = np.float32(-np.inf)

_CONSTS = {}


def _noise_t():
    def make():
        e = jax.random.exponential(jax.random.key(42), (_R, _V), dtype=jnp.float32)
        return jnp.minimum(-jnp.log(e), jnp.float32(3e37))

    if "n" not in _CONSTS:
        try:
            with jax.ensure_compile_time_eval():
                n = np.ascontiguousarray(np.asarray(make()).T)
                _CONSTS["n"] = jnp.asarray(n)
        except Exception:
            return make().T
    return _CONSTS["n"]


def _body(x_hbm, temps_hbm, noise_hbm, ov_hbm, oi_hbm,
          tbuf, lbuf0, lbuf1, nbuf0, nbuf1, vstage, istage, pvstage, pistage,
          sh_v, sh_i, lsem0, lsem1, nsem0, nsem1):
    cid = lax.axis_index("c")
    sid = lax.axis_index("s")
    wid = cid * 16 + sid
    pltpu.sync_copy(temps_hbm, tbuf)
    tvec = [tbuf[pl.ds(16 * k, 16)] for k in range(8)]

    start = 390 * wid + jnp.minimum(wid, 20)
    lbufs, nbufs = (lbuf0, lbuf1), (nbuf0, nbuf1)
    lsems, nsems = (lsem0, lsem1), (nsem0, nsem1)

    def fire(i):
        s = i & 1
        t0 = jnp.minimum(start + _CTILE * i, _NT - _CTILE)
        v0 = pl.multiple_of(t0 * 8, 8)
        return (pltpu.async_copy(x_hbm.at[pl.ds(v0, _CVOC)], lbufs[s], lsems[s]),
                pltpu.async_copy(noise_hbm.at[pl.ds(v0, _CVOC)], nbufs[s], nsems[s]),
                t0)

    best = [jnp.full((16,), _NINF, jnp.float32) for _ in range(8)]
    ci = [jnp.zeros((16,), jnp.int32) for _ in range(8)]

    pend = fire(0)
    for i in range(_NCHUNK):
        nxt = fire(i + 1) if i + 1 < _NCHUNK else None
        pend[0].wait()
        pend[1].wait()
        s = i & 1
        lb, nb = lbufs[s], nbufs[s]
        vbase = pend[2] * 8

        @plsc.parallel_loop(0, _CVOC, 1, unroll=2, carry=tuple(best) + tuple(ci))
        def scan(v, carry, lb=lb, nb=nb, vbase=vbase):
            acc = list(carry)
            vg = jnp.full((16,), vbase + v, jnp.int32)
            for k in range(8):
                val = lb[v, pl.ds(16 * k, 16)] + tvec[k] * nb[v, pl.ds(16 * k, 16)]
                m = val > acc[k]
                acc[k] = jnp.where(m, val, acc[k])
                acc[8 + k] = jnp.where(m, vg, acc[8 + k])
            return tuple(acc)

        best, ci = list(scan[:8]), list(scan[8:])
        pend = nxt

    for k in range(8):
        vstage[pl.ds(16 * k, 16)] = best[k]
        istage[pl.ds(16 * k, 16)] = ci[k]
    pltpu.sync_copy(vstage, sh_v.at[pl.ds(sid * 128, 128)])
    pltpu.sync_copy(istage, sh_i.at[pl.ds(sid * 128, 128)])
    plsc.subcore_barrier()

    @pl.when(sid == 0)
    def _():
        mv = [jnp.full((16,), _NINF, jnp.float32) for _ in range(8)]
        mi = [jnp.zeros((16,), jnp.int32) for _ in range(8)]
        for w in range(16):
            pltpu.sync_copy(sh_v.at[pl.ds(w * 128, 128)], pvstage)
            pltpu.sync_copy(sh_i.at[pl.ds(w * 128, 128)], pistage)
            for k in range(8):
                bv = pvstage[pl.ds(16 * k, 16)]
                bi = pistage[pl.ds(16 * k, 16)]
                take = (bv > mv[k]) | ((bv == mv[k]) & (bi < mi[k]))
                mv[k] = jnp.where(take, bv, mv[k])
                mi[k] = jnp.where(take, bi, mi[k])
        for k in range(8):
            vstage[pl.ds(16 * k, 16)] = mv[k]
            istage[pl.ds(16 * k, 16)] = mi[k]
        pltpu.sync_copy(vstage, ov_hbm.at[pl.ds(cid * 128, 128)])
        pltpu.sync_copy(istage, oi_hbm.at[pl.ds(cid * 128, 128)])


@jax.jit
def _sampler(x_t, temps, noise_t):
    mesh = plsc.VectorSubcoreMesh(core_axis_name="c", subcore_axis_name="s")
    k = pl.kernel(
        _body,
        out_type=(jax.ShapeDtypeStruct((2 * _R,), jnp.float32),
                  jax.ShapeDtypeStruct((2 * _R,), jnp.int32)),
        mesh=mesh,
        compiler_params=pltpu.CompilerParams(needs_layout_passes=False),
        scratch_types=[
            pltpu.VMEM((_R,), jnp.float32),
            pltpu.VMEM((_CVOC, _R), jnp.float32),
            pltpu.VMEM((_CVOC, _R), jnp.float32),
            pltpu.VMEM((_CVOC, _R), jnp.float32),
            pltpu.VMEM((_CVOC, _R), jnp.float32),
            pltpu.VMEM((_R,), jnp.float32),
            pltpu.VMEM((_R,), jnp.int32),
            pltpu.VMEM((_R,), jnp.float32),
            pltpu.VMEM((_R,), jnp.int32),
            pltpu.VMEM_SHARED((16 * _R,), jnp.float32),
            pltpu.VMEM_SHARED((16 * _R,), jnp.int32),
            pltpu.SemaphoreType.DMA,
            pltpu.SemaphoreType.DMA,
            pltpu.SemaphoreType.DMA,
            pltpu.SemaphoreType.DMA,
        ],
    )
    vals, idxs = k(x_t, temps, noise_t)
    v0, v1 = vals[:_R], vals[_R:]
    i0, i1 = idxs[:_R], idxs[_R:]
    take1 = (v1 > v0) | ((v1 == v0) & (i1 < i0))
    return jnp.where(take1, i1, i0)


def kernel(logits, temperatures):
    return _sampler(logits.T, temperatures, _noise_t())
